# baseline (speedup 1.0000x reference)
import functools

import jax
import jax.numpy as jnp
from jax import lax
from jax.experimental import pallas as pl
from jax.experimental.pallas import tpu as pltpu
from jax.experimental.pallas import tpu_sc as plsc

B, LD, LQ, V, D, NF = 4096, 200, 20, 1000000, 64, 4
W = NF + D
NC, NS = 2, 16
NW = NC * NS
CH = 512
KI = CH // 128
R1 = B * LD
R2 = B * LQ
LT, BT = LD // 8, B // 128
NH = 2
Q1 = LT * BT * NH // NW
Q2 = R2 // CH // NW

_mesh = plsc.VectorSubcoreMesh(core_axis_name="c", subcore_axis_name="s")


@functools.partial(
    pl.kernel,
    mesh=_mesh,
    compiler_params=pltpu.CompilerParams(use_tc_tiling_on_sc=False,
                                         needs_layout_passes=False),
    out_type=[
        jax.ShapeDtypeStruct((B, LD, D), jnp.float32),
        jax.ShapeDtypeStruct((R2 // CH, KI, 128, D), jnp.float32),
    ],
    scratch_types=[
        pltpu.VMEM((2, KI, 128), jnp.int32),
        pltpu.VMEM((2, KI, 128, D), jnp.float32),
        pltpu.SemaphoreType.DMA,
        pltpu.SemaphoreType.DMA,
    ],
)
def _emb_gather(x1_hbm, x2_hbm, tbl_hbm, out1_hbm, out2_hbm,
                idx_v, row_v, sem0, sem1):
    wid = lax.axis_index("s") * NC + lax.axis_index("c")
    sems = (sem0, sem1)

    def x1_fire(q, p):
        lt = q // (BT * NH)
        bt = (q // NH) % BT
        half = q % NH
        pltpu.sync_copy(x1_hbm.at[lt, bt, pl.ds(half * KI, KI)], idx_v.at[p])
        for j in range(KI):
            pltpu.async_copy(
                tbl_hbm.at[idx_v.at[p, j]],
                row_v.at[p, j],
                sems[p],
            )

    def drain(p):
        for j in range(KI):
            pltpu.make_async_copy(
                tbl_hbm.at[idx_v.at[p, j]],
                row_v.at[p, j],
                sems[p],
            ).wait()

    def x1_finish(q, p):
        lt = q // (BT * NH)
        bt = (q // NH) % BT
        half = q % NH
        l0 = lt * 8 + half * KI
        b0 = bt * 128
        drain(p)
        for j in range(KI):
            pltpu.sync_copy(row_v.at[p, j],
                            out1_hbm.at[pl.ds(b0, 128), l0 + j])

    def x2_fire(q, p):
        pltpu.sync_copy(x2_hbm.at[q], idx_v.at[p])
        for j in range(KI):
            pltpu.async_copy(
                tbl_hbm.at[idx_v.at[p, j]],
                row_v.at[p, j],
                sems[p],
            )

    def x2_finish(q, p):
        drain(p)
        pltpu.sync_copy(row_v.at[p], out2_hbm.at[q])

    def sweep(fire, finish, q0, nq):
        fire(q0, 0)

        def pair(t, carry):
            q = q0 + 2 * t
            fire(q + 1, 1)
            finish(q, 0)

            @pl.when(q + 2 < q0 + nq)
            def _():
                fire(q + 2, 0)

            finish(q + 1, 1)
            return carry

        lax.fori_loop(0, nq // 2, pair, 0)

        @pl.when(nq % 2 == 1)
        def _():
            finish(q0 + nq - 1, 0)

    sweep(x1_fire, x1_finish, wid * Q1, Q1)
    sweep(x2_fire, x2_finish, wid * Q2, Q2)


def kernel(x1, x1_f, x1_pos, x1_ner, x1_mask, x2, x2_mask, sent_lens, emb_table):
    del x1_pos, x1_ner, x1_mask, x2_mask, sent_lens
    x1_tiled = x1.reshape(BT, 128, LT, 8).transpose(2, 0, 3, 1)
    x2_chunks = x2.reshape(R2 // CH, KI, 128)
    e1, e2 = _emb_gather(x1_tiled, x2_chunks, emb_table)
    x1_all = jnp.concatenate([x1_f, e1], axis=-1)
    return x1_all, e2.reshape(B, LQ, D)


# speedup vs baseline: 1.6289x; 1.6289x over previous
import functools

import jax
import jax.numpy as jnp
from jax import lax
from jax.experimental import pallas as pl
from jax.experimental.pallas import tpu as pltpu
from jax.experimental.pallas import tpu_sc as plsc

B, LD, LQ, V, D, NF = 4096, 200, 20, 1000000, 64, 4
W = NF + D
NC, NS = 2, 16
NW = NC * NS
CH = 512
KI = CH // 128
R1 = B * LD
R2 = B * LQ
Q1 = R1 // CH // NW
Q2 = R2 // CH // NW

_mesh = plsc.VectorSubcoreMesh(core_axis_name="c", subcore_axis_name="s")


def _gather_body(idx_hbm, tbl_hbm, out_hbm, idx_v, row_v, sem0, sem1, nq):
    wid = lax.axis_index("s") * NC + lax.axis_index("c")
    sems = (sem0, sem1)

    def fire(r0, p):
        pltpu.sync_copy(idx_hbm.at[pl.ds(r0, CH)], idx_v.at[p])
        for j in range(KI):
            pltpu.async_copy(
                tbl_hbm.at[idx_v.at[p, pl.ds(j * 128, 128)]],
                row_v.at[p, pl.ds(j * 128, 128)],
                sems[p],
            )

    def finish(q, p):
        for j in range(KI):
            pltpu.make_async_copy(
                tbl_hbm.at[idx_v.at[p, pl.ds(j * 128, 128)]],
                row_v.at[p, pl.ds(j * 128, 128)],
                sems[p],
            ).wait()
        pltpu.sync_copy(row_v.at[p], out_hbm.at[pl.ds(q * CH, CH)])

    q0 = wid * nq
    fire(q0 * CH, 0)

    def pair(t, carry):
        q = q0 + 2 * t
        fire((q + 1) * CH, 1)
        finish(q, 0)

        @pl.when(q + 2 < q0 + nq)
        def _():
            fire((q + 2) * CH, 0)

        finish(q + 1, 1)
        return carry

    lax.fori_loop(0, nq // 2, pair, 0)

    @pl.when(nq % 2 == 1)
    def _():
        finish(q0 + nq - 1, 0)


def _sc_gather(nrows, nq):
    return functools.partial(
        pl.kernel,
        mesh=_mesh,
        compiler_params=pltpu.CompilerParams(use_tc_tiling_on_sc=False,
                                             needs_layout_passes=False),
        out_type=jax.ShapeDtypeStruct((nrows, D), jnp.float32),
        scratch_types=[
            pltpu.VMEM((2, CH), jnp.int32),
            pltpu.VMEM((2, CH, D), jnp.float32),
            pltpu.SemaphoreType.DMA,
            pltpu.SemaphoreType.DMA,
        ],
    )(functools.partial(_gather_body, nq=nq))


_gather_x1 = _sc_gather(R1, Q1)
_gather_x2 = _sc_gather(R2, Q2)


def kernel(x1, x1_f, x1_pos, x1_ner, x1_mask, x2, x2_mask, sent_lens, emb_table):
    del x1_pos, x1_ner, x1_mask, x2_mask, sent_lens
    e1 = _gather_x1(x1.reshape(R1), emb_table)
    e2 = _gather_x2(x2.reshape(R2), emb_table)
    x1_all = jnp.concatenate([x1_f, e1.reshape(B, LD, D)], axis=-1)
    return x1_all, e2.reshape(B, LQ, D)
